# baseline (speedup 1.0000x reference)
import dataclasses
import functools

import jax
import jax.numpy as jnp
from jax import lax
from jax.experimental import pallas as pl
from jax.experimental.pallas import tpu as pltpu
from jax.experimental.pallas import tpu_sc as plsc

N_NODES = 10000
N_EDGES = 320000
D = 128

NC = 2
NS = 16
NW = NC * NS
E_PER_W = N_EDGES // NW
CHUNK = 80
N_CHUNK = E_PER_W // CHUNK
HALF = 5120
ACC_ROWS = HALF + 128
ZERO_PER_TILE = ACC_ROWS // NS
DUMP_PER_TILE = HALF // NS
OUT_ROWS = 2 * HALF

CAP = 5440
E0P = NW * CAP
EP = 2 * E0P
T_ROWS = E0P // NS
FN = T_ROWS // CHUNK

_mesh = plsc.VectorSubcoreMesh(core_axis_name="c", subcore_axis_name="s")

_sc_params = pltpu.CompilerParams()
if "needs_layout_passes" in pltpu.CompilerParams.__dataclass_fields__:
    _sc_params = dataclasses.replace(_sc_params, needs_layout_passes=False)


def _pipeline2(n, stage_in, wait_in, stage_out, wait_out):
    def body(i, b):
        wait_in(b)
        stage_out(i, b)

    stage_in(0, 0)
    stage_in(1, 1)
    n_even = n - (n % 2)

    @pl.loop(0, max(n_even - 2, 0), step=2)
    def _(i):
        body(i, 0)
        wait_out(0)
        stage_in(i + 2, 0)
        body(i + 1, 1)
        wait_out(1)
        stage_in(i + 3, 1)

    if n % 2:
        body(n - 3, 0)
        wait_out(0)
        stage_in(n - 1, 0)
        body(n - 2, 1)
        body(n - 1, 0)
    else:
        body(n - 2, 0)
        body(n - 1, 1)
    wait_out(0)
    wait_out(1)


CAPB = CAP + 16

@functools.partial(
    pl.kernel, mesh=_mesh, compiler_params=_sc_params,
    out_type=[
        jax.ShapeDtypeStruct((EP,), jnp.int32),
        jax.ShapeDtypeStruct((EP,), jnp.int32),
        jax.ShapeDtypeStruct((EP,), jnp.int32),
    ],
    scratch_types=[
        pltpu.VMEM((E_PER_W,), jnp.int32),
        pltpu.VMEM((E_PER_W,), jnp.int32),
        pltpu.VMEM((CAPB,), jnp.int32),
        pltpu.VMEM((CAPB,), jnp.int32),
        pltpu.VMEM((CAPB,), jnp.int32),
        pltpu.VMEM((CAPB,), jnp.int32),
        pltpu.VMEM((CAPB,), jnp.int32),
        pltpu.VMEM((CAPB,), jnp.int32),
    ],
)
def _sc_partition(src_hbm, dst_hbm, srcp_hbm, dstp_hbm, idp_hbm,
                  src_v, dst_v, sb_v, db_v, ib_v, sa_v, da_v, ia_v):
    wid = lax.axis_index("s") * NC + lax.axis_index("c")
    ebase = wid * E_PER_W
    pltpu.sync_copy(src_hbm.at[pl.ds(ebase, E_PER_W)], src_v)
    pltpu.sync_copy(dst_hbm.at[pl.ds(ebase, E_PER_W)], dst_v)

    zero16 = jnp.zeros((16,), jnp.int32)
    sent = HALF + lax.iota(jnp.int32, 16) + 16 * (wid % 8)

    @pl.loop(0, CAPB, step=16)
    def _(r):
        s = pl.ds(r, 16)
        sb_v[s] = zero16
        sa_v[s] = zero16
        ib_v[s] = zero16
        ia_v[s] = zero16
        db_v[s] = sent
        da_v[s] = sent

    def jbody(v, cur):
        cur_b, cur_a = cur
        d = dst_v[pl.ds(v * 16, 16)]
        s = src_v[pl.ds(v * 16, 16)]
        ids = ebase + v * 16 + lax.iota(jnp.int32, 16)
        below = d < HALF
        above = jnp.logical_not(below)
        plsc.store_compressed(sb_v.at[pl.ds(cur_b, 16)], s, mask=below)
        plsc.store_compressed(db_v.at[pl.ds(cur_b, 16)], d, mask=below)
        plsc.store_compressed(ib_v.at[pl.ds(cur_b, 16)], ids, mask=below)
        plsc.store_compressed(sa_v.at[pl.ds(cur_a, 16)], s, mask=above)
        plsc.store_compressed(da_v.at[pl.ds(cur_a, 16)], d - HALF, mask=above)
        plsc.store_compressed(ia_v.at[pl.ds(cur_a, 16)], ids, mask=above)
        nb = jnp.sum(jnp.where(below, 1, 0).astype(jnp.int32))
        return (cur_b + nb, cur_a + (16 - nb))

    lax.fori_loop(0, E_PER_W // 16, jbody, (jnp.int32(0), jnp.int32(0)))

    below_base = wid * CAP
    above_base = E0P + wid * CAP
    pltpu.sync_copy(sb_v.at[pl.ds(0, CAP)], srcp_hbm.at[pl.ds(below_base, CAP)])
    pltpu.sync_copy(db_v.at[pl.ds(0, CAP)], dstp_hbm.at[pl.ds(below_base, CAP)])
    pltpu.sync_copy(ib_v.at[pl.ds(0, CAP)], idp_hbm.at[pl.ds(below_base, CAP)])
    pltpu.sync_copy(sa_v.at[pl.ds(0, CAP)], srcp_hbm.at[pl.ds(above_base, CAP)])
    pltpu.sync_copy(da_v.at[pl.ds(0, CAP)], dstp_hbm.at[pl.ds(above_base, CAP)])
    pltpu.sync_copy(ia_v.at[pl.ds(0, CAP)], idp_hbm.at[pl.ds(above_base, CAP)])


@functools.partial(
    pl.kernel, mesh=_mesh,
    out_type=jax.ShapeDtypeStruct((OUT_ROWS, D), jnp.float32),
    scratch_types=[
        pltpu.VMEM((T_ROWS,), jnp.int32),
        pltpu.VMEM((FN, CHUNK), jnp.int32),
        pltpu.VMEM((T_ROWS,), jnp.int32),
        pltpu.VMEM((2, CHUNK, D), jnp.float32),
        pltpu.VMEM((2, CHUNK, D), jnp.float32),
        pltpu.SemaphoreType.DMA,
        pltpu.SemaphoreType.DMA,
        pltpu.SemaphoreType.DMA,
        pltpu.SemaphoreType.DMA,
        pltpu.VMEM_SHARED((ACC_ROWS, D), jnp.float32),
    ],
)
def _sc_fused(table_hbm, srci_hbm, dsti_hbm, idi_hbm, e_hbm, zeros_hbm,
              out_hbm, si_v, di_v, ii_v, e_v, g_v, sin0, sin1, sout0, sout1,
              accum):
    cid = lax.axis_index("c")
    sid = lax.axis_index("s")
    rbase = cid * E0P + sid * T_ROWS
    cbase = cid * (E0P // CHUNK) + sid * FN
    pltpu.sync_copy(srci_hbm.at[pl.ds(rbase, T_ROWS)], si_v)
    pltpu.sync_copy(dsti_hbm.at[pl.ds(cbase, FN)], di_v)
    pltpu.sync_copy(idi_hbm.at[pl.ds(rbase, T_ROWS)], ii_v)
    pltpu.sync_copy(zeros_hbm, accum.at[pl.ds(sid * ZERO_PER_TILE,
                                              ZERO_PER_TILE)])
    plsc.subcore_barrier()
    sin = (sin0, sin1)
    sout = (sout0, sout1)

    def stage_in(i, b):
        off = pl.multiple_of(i * CHUNK, CHUNK)
        pltpu.async_copy(e_hbm.at[ii_v.at[pl.ds(off, CHUNK)]],
                         e_v.at[b], sin[b])
        pltpu.async_copy(table_hbm.at[si_v.at[pl.ds(off, CHUNK)]],
                         g_v.at[b], sin[b])

    def wait_in(b):
        pltpu.make_async_copy(e_hbm.at[pl.ds(0, CHUNK)], e_v.at[b],
                              sin[b]).wait()
        pltpu.make_async_copy(e_hbm.at[pl.ds(0, CHUNK)], g_v.at[b],
                              sin[b]).wait()

    def stage_out(i, b):
        @pl.loop(0, CHUNK)
        def _(r):
            for q in range(D // 16):
                s = pl.ds(q * 16, 16)
                g_v[b, r, s] = jnp.maximum(g_v[b, r, s] + e_v[b, r, s], 0.0)

        pltpu.async_copy(g_v.at[b], accum.at[di_v.at[i]], sout[b], add=True)

    def wait_out(b):
        pltpu.make_async_copy(g_v.at[b], accum.at[di_v.at[0]],
                              sout[b]).wait()

    _pipeline2(FN, stage_in, wait_in, stage_out, wait_out)

    plsc.subcore_barrier()
    pltpu.sync_copy(
        accum.at[pl.ds(sid * DUMP_PER_TILE, DUMP_PER_TILE)],
        out_hbm.at[pl.ds(cid * HALF + sid * DUMP_PER_TILE, DUMP_PER_TILE)],
    )


def _e_body(ea_ref, w_ref, b_ref, out_ref):
    out_ref[...] = jnp.dot(ea_ref[...], w_ref[...],
                           preferred_element_type=jnp.float32,
                           precision=lax.Precision.HIGHEST) + b_ref[...]


def _tc_e(ea, w_et, b_e):
    blk = 2000
    return pl.pallas_call(
        _e_body,
        grid=(N_EDGES // blk,),
        in_specs=[
            pl.BlockSpec((blk, 16), lambda i: (i, 0)),
            pl.BlockSpec((16, D), lambda i: (0, 0)),
            pl.BlockSpec((1, D), lambda i: (0, 0)),
        ],
        out_specs=pl.BlockSpec((blk, D), lambda i: (i, 0)),
        out_shape=jax.ShapeDtypeStruct((N_EDGES, D), jnp.float32),
    )(ea, w_et, b_e)


def _node_body(h_ref, p_ref, w1_ref, b1_ref, w2_ref, b2_ref,
               out_ref, *, relu_out):
    z = h_ref[...] + p_ref[...]
    t = jnp.maximum(
        jnp.dot(z, w1_ref[...], preferred_element_type=jnp.float32,
                precision=lax.Precision.HIGHEST) + b1_ref[...], 0.0)
    o = jnp.dot(t, w2_ref[...], preferred_element_type=jnp.float32,
                precision=lax.Precision.HIGHEST) + b2_ref[...]
    if relu_out:
        o = jnp.maximum(o, 0.0)
    out_ref[...] = o


_NODE_SPECS = [
    pl.BlockSpec((1000, D), lambda i: (i, 0)),
    pl.BlockSpec((1000, D), lambda i: (i, 0)),
    pl.BlockSpec((D, D), lambda i: (0, 0)),
    pl.BlockSpec((1, D), lambda i: (0, 0)),
    pl.BlockSpec((D, D), lambda i: (0, 0)),
    pl.BlockSpec((1, D), lambda i: (0, 0)),
]


def _tc_node(h, p, w1t, b1, w2t, b2, relu_out):
    return pl.pallas_call(
        functools.partial(_node_body, relu_out=relu_out),
        grid=(N_NODES // 1000,),
        in_specs=_NODE_SPECS,
        out_specs=pl.BlockSpec((1000, D), lambda i: (i, 0)),
        out_shape=jax.ShapeDtypeStruct((N_NODES, D), jnp.float32),
    )(h, p, w1t, b1, w2t, b2)


def _node_mean_body(h_ref, p_ref, w1_ref, b1_ref, w2_ref, b2_ref, out_ref):
    i = pl.program_id(0)
    z = h_ref[...] + p_ref[...]
    t = jnp.maximum(
        jnp.dot(z, w1_ref[...], preferred_element_type=jnp.float32,
                precision=lax.Precision.HIGHEST) + b1_ref[...], 0.0)
    o = jnp.dot(t, w2_ref[...], preferred_element_type=jnp.float32,
                precision=lax.Precision.HIGHEST) + b2_ref[...]

    @pl.when(i == 0)
    def _():
        out_ref[...] = jnp.zeros_like(out_ref)

    out_ref[...] += jnp.sum(o, axis=0, keepdims=True) * (1.0 / N_NODES)


def _tc_node_mean(h, p, w1t, b1, w2t, b2):
    return pl.pallas_call(
        _node_mean_body,
        grid=(N_NODES // 1000,),
        in_specs=_NODE_SPECS,
        out_specs=pl.BlockSpec((1, D), lambda i: (0, 0)),
        out_shape=jax.ShapeDtypeStruct((1, D), jnp.float32),
    )(h, p, w1t, b1, w2t, b2)


def kernel(x, edge_index, edge_attr,
           W_e0, b_e0, W1_0, b1_0, W2_0, b2_0,
           W_e1, b_e1, W1_1, b1_1, W2_1, b2_1,
           W_e2, b_e2, W1_2, b1_2, W2_2, b2_2):
    src = jnp.asarray(edge_index[0], jnp.int32)
    dst = jnp.asarray(edge_index[1], jnp.int32)
    zeros = jnp.zeros((ZERO_PER_TILE, D), jnp.float32)

    src_p, dst_p, id_p = _sc_partition(src, dst)
    dsti = dst_p.reshape(EP // CHUNK, CHUNK)

    params = [
        (W_e0, b_e0, W1_0, b1_0, W2_0, b2_0),
        (W_e1, b_e1, W1_1, b1_1, W2_1, b2_1),
        (W_e2, b_e2, W1_2, b1_2, W2_2, b2_2),
    ]
    e_ps = [_tc_e(edge_attr, w_e.T, b_e.reshape(1, D))
            for (w_e, b_e, _, _, _, _) in params]
    h = x
    for l, (w_e, b_e, w1, b1, w2, b2) in enumerate(params):
        p = _sc_fused(h, src_p, dsti, id_p, e_ps[l], zeros)
        if l < 2:
            h = _tc_node(h, p, w1.T, b1.reshape(1, D),
                         w2.T, b2.reshape(1, D), relu_out=True)
        else:
            h = _tc_node_mean(h, p, w1.T, b1.reshape(1, D),
                              w2.T, b2.reshape(1, D))
    return h


# speedup vs baseline: 2.4332x; 2.4332x over previous
import functools

import jax
import jax.numpy as jnp
from jax import lax
from jax.experimental import pallas as pl
from jax.experimental.pallas import tpu as pltpu
from jax.experimental.pallas import tpu_sc as plsc

N_NODES = 10000
N_EDGES = 320000
D = 128

NC = 2
NS = 16
NW = NC * NS
E_PER_W = N_EDGES // NW
CHUNK = 80
N_CHUNK = E_PER_W // CHUNK
K = 5
SUPER = CHUNK * K
N_SUPER = E_PER_W // SUPER
HALF = 5120
ACC_ROWS = HALF + 128
ZERO_PER_TILE = ACC_ROWS // NS
DUMP_PER_TILE = HALF // NS
E_PER_TILE = N_EDGES // NS
N_CHUNK_SC = E_PER_TILE // CHUNK
K_SC = 2
SUPER_SC = CHUNK * K_SC
N_SUPER_SC = E_PER_TILE // SUPER_SC
OUT_ROWS = 2 * HALF

_mesh = plsc.VectorSubcoreMesh(core_axis_name="c", subcore_axis_name="s")


def _pipeline2(n, stage_in, wait_in, stage_out, wait_out):
    def body(i, b):
        wait_in(b)
        stage_out(i, b)

    stage_in(0, 0)
    stage_in(1, 1)
    n_even = n - (n % 2)

    @pl.loop(0, max(n_even - 2, 0), step=2)
    def _(i):
        body(i, 0)
        wait_out(0)
        stage_in(i + 2, 0)
        body(i + 1, 1)
        wait_out(1)
        stage_in(i + 3, 1)

    if n % 2:
        body(n - 3, 0)
        wait_out(0)
        stage_in(n - 1, 0)
        body(n - 2, 1)
        body(n - 1, 0)
    else:
        body(n - 2, 0)
        body(n - 1, 1)
    wait_out(0)
    wait_out(1)


@functools.partial(
    pl.kernel, mesh=_mesh,
    out_type=jax.ShapeDtypeStruct((N_EDGES, D), jnp.float32),
    scratch_types=[
        pltpu.VMEM((N_CHUNK, CHUNK), jnp.int32),
        pltpu.VMEM((2, SUPER, D), jnp.float32),
        pltpu.SemaphoreType.DMA,
        pltpu.SemaphoreType.DMA,
        pltpu.SemaphoreType.DMA,
        pltpu.SemaphoreType.DMA,
    ],
)
def _sc_gather(table_hbm, idx_hbm, out_hbm, idx_v, rows_v, sg0, sg1, so0, so1):
    wid = lax.axis_index("s") * NC + lax.axis_index("c")
    base = wid * E_PER_W
    pltpu.sync_copy(idx_hbm.at[wid], idx_v)
    sg = (sg0, sg1)
    so = (so0, so1)

    def stage_in(i, b):
        for t in range(K):
            pltpu.async_copy(
                table_hbm.at[idx_v.at[i * K + t]],
                rows_v.at[b].at[pl.ds(t * CHUNK, CHUNK)],
                sg[b],
            )

    def wait_in(b):
        for _ in range(K):
            pltpu.make_async_copy(
                table_hbm.at[idx_v.at[0]],
                rows_v.at[b].at[pl.ds(0, CHUNK)],
                sg[b],
            ).wait()

    def stage_out(i, b):
        pltpu.async_copy(
            rows_v.at[b], out_hbm.at[pl.ds(base + i * SUPER, SUPER)], so[b],
        )

    def wait_out(b):
        pltpu.make_async_copy(
            rows_v.at[b], out_hbm.at[pl.ds(0, SUPER)], so[b],
        ).wait()

    _pipeline2(N_SUPER, stage_in, wait_in, stage_out, wait_out)


@functools.partial(
    pl.kernel, mesh=_mesh,
    out_type=jax.ShapeDtypeStruct((OUT_ROWS, D), jnp.float32),
    scratch_types=[
        pltpu.VMEM((N_CHUNK_SC, CHUNK), jnp.int32),
        pltpu.VMEM((2, SUPER_SC, D), jnp.float32),
        pltpu.SemaphoreType.DMA,
        pltpu.SemaphoreType.DMA,
        pltpu.SemaphoreType.DMA,
        pltpu.SemaphoreType.DMA,
        pltpu.VMEM_SHARED((ACC_ROWS, D), jnp.float32),
    ],
)
def _sc_scatter_add(msg_hbm, idx_hbm, zeros_hbm, out_hbm,
                    idx_v, upd_v, sm0, sm1, ss0, ss1, accum):
    cid = lax.axis_index("c")
    sid = lax.axis_index("s")
    base = cid * HALF
    pltpu.sync_copy(idx_hbm.at[sid], idx_v)
    pltpu.sync_copy(zeros_hbm, accum.at[pl.ds(sid * ZERO_PER_TILE,
                                              ZERO_PER_TILE)])

    @pl.loop(0, N_CHUNK_SC)
    def _(j):
        for q in range(CHUNK // 16):
            v = idx_v[j, pl.ds(q * 16, 16)] - base
            inb = (v >= 0) & (v < HALF)
            dummy = jnp.full((16,), HALF + sid * 8 + q, jnp.int32)
            idx_v[j, pl.ds(q * 16, 16)] = jnp.where(inb, v, dummy)

    plsc.subcore_barrier()
    sm = (sm0, sm1)
    ss = (ss0, ss1)

    def stage_in(i, b):
        pltpu.async_copy(
            msg_hbm.at[pl.ds(sid * E_PER_TILE + i * SUPER_SC, SUPER_SC)],
            upd_v.at[b], sm[b],
        )

    def wait_in(b):
        pltpu.make_async_copy(
            msg_hbm.at[pl.ds(0, SUPER_SC)], upd_v.at[b], sm[b],
        ).wait()

    def stage_out(i, b):
        for t in range(K_SC):
            pltpu.async_copy(
                upd_v.at[b].at[pl.ds(t * CHUNK, CHUNK)],
                accum.at[idx_v.at[i * K_SC + t]],
                ss[b], add=True,
            )

    def wait_out(b):
        for _ in range(K_SC):
            pltpu.make_async_copy(
                upd_v.at[b].at[pl.ds(0, CHUNK)],
                accum.at[idx_v.at[0]],
                ss[b],
            ).wait()

    _pipeline2(N_SUPER_SC, stage_in, wait_in, stage_out, wait_out)

    plsc.subcore_barrier()
    pltpu.sync_copy(
        accum.at[pl.ds(sid * DUMP_PER_TILE, DUMP_PER_TILE)],
        out_hbm.at[pl.ds(base + sid * DUMP_PER_TILE, DUMP_PER_TILE)],
    )


def _msg_body(g_ref, ea_ref, w_ref, b_ref, out_ref):
    e = jnp.dot(ea_ref[...], w_ref[...],
                preferred_element_type=jnp.float32,
                precision=lax.Precision.HIGHEST)
    out_ref[...] = jnp.maximum(g_ref[...] + e + b_ref[...], 0.0)


def _tc_msg(g, ea, w_et, b_e):
    blk = 2000
    return pl.pallas_call(
        _msg_body,
        grid=(N_EDGES // blk,),
        in_specs=[
            pl.BlockSpec((blk, D), lambda i: (i, 0)),
            pl.BlockSpec((blk, 16), lambda i: (i, 0)),
            pl.BlockSpec((16, D), lambda i: (0, 0)),
            pl.BlockSpec((1, D), lambda i: (0, 0)),
        ],
        out_specs=pl.BlockSpec((blk, D), lambda i: (i, 0)),
        out_shape=jax.ShapeDtypeStruct((N_EDGES, D), jnp.float32),
    )(g, ea, w_et, b_e)


def _node_body(h_ref, p_ref, w1_ref, b1_ref, w2_ref, b2_ref,
               out_ref, *, relu_out):
    z = h_ref[...] + p_ref[...]
    t = jnp.maximum(
        jnp.dot(z, w1_ref[...], preferred_element_type=jnp.float32,
                precision=lax.Precision.HIGHEST) + b1_ref[...], 0.0)
    o = jnp.dot(t, w2_ref[...], preferred_element_type=jnp.float32,
                precision=lax.Precision.HIGHEST) + b2_ref[...]
    if relu_out:
        o = jnp.maximum(o, 0.0)
    out_ref[...] = o


_NODE_SPECS = [
    pl.BlockSpec((1000, D), lambda i: (i, 0)),
    pl.BlockSpec((1000, D), lambda i: (i, 0)),
    pl.BlockSpec((D, D), lambda i: (0, 0)),
    pl.BlockSpec((1, D), lambda i: (0, 0)),
    pl.BlockSpec((D, D), lambda i: (0, 0)),
    pl.BlockSpec((1, D), lambda i: (0, 0)),
]


def _tc_node(h, p, w1t, b1, w2t, b2, relu_out):
    return pl.pallas_call(
        functools.partial(_node_body, relu_out=relu_out),
        grid=(N_NODES // 1000,),
        in_specs=_NODE_SPECS,
        out_specs=pl.BlockSpec((1000, D), lambda i: (i, 0)),
        out_shape=jax.ShapeDtypeStruct((N_NODES, D), jnp.float32),
    )(h, p, w1t, b1, w2t, b2)


def _node_mean_body(h_ref, p_ref, w1_ref, b1_ref, w2_ref, b2_ref, out_ref):
    i = pl.program_id(0)
    z = h_ref[...] + p_ref[...]
    t = jnp.maximum(
        jnp.dot(z, w1_ref[...], preferred_element_type=jnp.float32,
                precision=lax.Precision.HIGHEST) + b1_ref[...], 0.0)
    o = jnp.dot(t, w2_ref[...], preferred_element_type=jnp.float32,
                precision=lax.Precision.HIGHEST) + b2_ref[...]

    @pl.when(i == 0)
    def _():
        out_ref[...] = jnp.zeros_like(out_ref)

    out_ref[...] += jnp.sum(o, axis=0, keepdims=True) * (1.0 / N_NODES)


def _tc_node_mean(h, p, w1t, b1, w2t, b2):
    return pl.pallas_call(
        _node_mean_body,
        grid=(N_NODES // 1000,),
        in_specs=_NODE_SPECS,
        out_specs=pl.BlockSpec((1, D), lambda i: (0, 0)),
        out_shape=jax.ShapeDtypeStruct((1, D), jnp.float32),
    )(h, p, w1t, b1, w2t, b2)


def kernel(x, edge_index, edge_attr,
           W_e0, b_e0, W1_0, b1_0, W2_0, b2_0,
           W_e1, b_e1, W1_1, b1_1, W2_1, b2_1,
           W_e2, b_e2, W1_2, b1_2, W2_2, b2_2):
    src3 = jnp.asarray(edge_index[0], jnp.int32).reshape(NW, N_CHUNK, CHUNK)
    dst3 = jnp.asarray(edge_index[1], jnp.int32).reshape(NS, N_CHUNK_SC, CHUNK)
    zeros = jnp.zeros((ZERO_PER_TILE, D), jnp.float32)

    params = [
        (W_e0, b_e0, W1_0, b1_0, W2_0, b2_0),
        (W_e1, b_e1, W1_1, b1_1, W2_1, b2_1),
        (W_e2, b_e2, W1_2, b1_2, W2_2, b2_2),
    ]
    h = x
    for l, (w_e, b_e, w1, b1, w2, b2) in enumerate(params):
        g = _sc_gather(h, src3)
        msg = _tc_msg(g, edge_attr, w_e.T, b_e.reshape(1, D))
        p = _sc_scatter_add(msg, dst3, zeros)
        if l < 2:
            h = _tc_node(h, p, w1.T, b1.reshape(1, D),
                         w2.T, b2.reshape(1, D), relu_out=True)
        else:
            h = _tc_node_mean(h, p, w1.T, b1.reshape(1, D),
                              w2.T, b2.reshape(1, D))
    return h
